# baseline (speedup 1.0000x reference)
import jax
import jax.numpy as jnp
from jax.experimental import pallas as pl
from jax.experimental.pallas import tpu as pltpu

_NUM_CLASSES = 400
_LABEL_SMOOTH = 0.1
_ALPHA = 1.0
_IGNORE_CLS = -1
_B = 16
_NBUF = 4
_TC = 8


def _mix_pairs_body(rowa_ref, rowb_ref, ca_s_ref, ca_o_ref, cb_s_ref, cb_o_ref,
                    xa_ref, xb_ref, o_ref, oa_buf, ob_buf, sem_a, sem_b):
    g0 = pl.program_id(0)
    g1 = pl.program_id(1)
    g2 = pl.program_id(2)
    n1 = pl.num_programs(1)
    n2 = pl.num_programs(2)
    step = (g0 * n1 + g1) * n2 + g2
    nsteps = pl.num_programs(0) * n1 * n2
    slot = step % _NBUF

    ra = rowa_ref[g0]
    rb = rowb_ref[g0]
    dst_a = o_ref.at[ra, g1, pl.ds(g2 * _TC, _TC)]
    dst_b = o_ref.at[rb, g1, pl.ds(g2 * _TC, _TC)]

    @pl.when(step >= _NBUF)
    def _():
        pltpu.make_async_copy(oa_buf.at[slot], dst_a, sem_a.at[slot]).wait()
        pltpu.make_async_copy(ob_buf.at[slot], dst_b, sem_b.at[slot]).wait()

    xa = xa_ref[0, 0]
    xb = xb_ref[0, 0]
    oa_buf[slot] = xa * ca_s_ref[g0] + xb * ca_o_ref[g0]
    ob_buf[slot] = xb * cb_s_ref[g0] + xa * cb_o_ref[g0]

    pltpu.make_async_copy(oa_buf.at[slot], dst_a, sem_a.at[slot]).start()
    pltpu.make_async_copy(ob_buf.at[slot], dst_b, sem_b.at[slot]).start()

    @pl.when(step == nsteps - 1)
    def _():
        for s in range(_NBUF):
            pltpu.make_async_copy(oa_buf.at[s], dst_a, sem_a.at[s]).wait()
            pltpu.make_async_copy(ob_buf.at[s], dst_b, sem_b.at[s]).wait()


def _mix_x(x, rowa, rowb, ca_s, ca_o, cb_s, cb_o):
    b, c, t, h, w = x.shape
    grid_spec = pltpu.PrefetchScalarGridSpec(
        num_scalar_prefetch=6,
        grid=(b // 2, c, t // _TC),
        in_specs=[
            pl.BlockSpec((1, 1, _TC, h, w),
                         lambda i, j, k, ra, rb, *_: (ra[i], j, k, 0, 0)),
            pl.BlockSpec((1, 1, _TC, h, w),
                         lambda i, j, k, ra, rb, *_: (rb[i], j, k, 0, 0)),
        ],
        out_specs=pl.BlockSpec(memory_space=pl.ANY),
        scratch_shapes=[
            pltpu.VMEM((_NBUF, _TC, h, w), jnp.float32),
            pltpu.VMEM((_NBUF, _TC, h, w), jnp.float32),
            pltpu.SemaphoreType.DMA((_NBUF,)),
            pltpu.SemaphoreType.DMA((_NBUF,)),
        ],
    )
    return pl.pallas_call(
        _mix_pairs_body,
        grid_spec=grid_spec,
        out_shape=jax.ShapeDtypeStruct(x.shape, jnp.float32),
        compiler_params=pltpu.CompilerParams(
            dimension_semantics=("arbitrary", "arbitrary", "arbitrary"),
        ),
    )(rowa, rowb, ca_s, ca_o, cb_s, cb_o, x, x)


def _one_hot_smooth(t):
    nt = _LABEL_SMOOTH / _NUM_CLASSES
    tv = 1.0 - _LABEL_SMOOTH + nt
    hot = jax.nn.one_hot(jnp.squeeze(t, axis=-1), _NUM_CLASSES, dtype=jnp.float32)
    return jnp.where(hot > 0.5, jnp.float32(tv), jnp.float32(nt))


def kernel(x_video_rgb, labels_action, labels_subclips_action):
    ts = jnp.squeeze(labels_subclips_action, axis=-1)
    mask = jnp.all(ts != _IGNORE_CLS, axis=-1)
    k = jnp.sum(mask)
    no_mix = k <= 1
    order = jnp.argsort(jnp.logical_not(mask), stable=True)
    rank = jnp.cumsum(mask) - 1
    partner = order[jnp.clip(k - 1 - rank, 0, _B - 1)].astype(jnp.int32)
    lam = jax.random.beta(jax.random.key(1), _ALPHA, _ALPHA)
    mix_on = mask & jnp.logical_not(no_mix)
    lam_rows = jnp.where(mix_on, lam, 1.0).astype(jnp.float32)

    idx = jnp.arange(_B, dtype=jnp.int32)
    partner_eff = jnp.where(mix_on, partner, idx)
    self_mix = mix_on & (partner_eff == idx)
    c_self = lam_rows + jnp.where(self_mix, 1.0 - lam, 0.0).astype(jnp.float32)
    c_oth = jnp.where(mix_on & (partner_eff != idx), 1.0 - lam, 0.0).astype(jnp.float32)
    is_single = partner_eff == idx
    singles_sorted = jnp.sort(jnp.where(is_single, idx, _B + 1))
    srank = (jnp.cumsum(is_single) - 1).astype(jnp.int32)
    mate = singles_sorted[jnp.clip(jnp.bitwise_xor(srank, 1), 0, _B - 1)]
    pair_to = jnp.where(is_single, mate, partner_eff).astype(jnp.int32)
    rowa = jnp.sort(jnp.where(idx < pair_to, idx, _B))[: _B // 2].astype(jnp.int32)
    rowb = pair_to[rowa]

    x_out = _mix_x(x_video_rgb, rowa, rowb,
                   c_self[rowa], c_oth[rowa], c_self[rowb], c_oth[rowb])

    labels_out = _one_hot_smooth(labels_action)
    subclips_ignore_index = labels_subclips_action == _IGNORE_CLS
    val_tmp = jnp.where(subclips_ignore_index, 0, labels_subclips_action)
    labels_subclips_out = _one_hot_smooth(val_tmp)

    lam_c = lam_rows[:, None]
    labels_out = lam_c * labels_out + (1.0 - lam_c) * labels_out[partner]
    lam_s = lam_rows[:, None, None]
    labels_subclips_out = (
        lam_s * labels_subclips_out + (1.0 - lam_s) * labels_subclips_out[partner]
    )
    return (x_out, labels_out, labels_subclips_out, subclips_ignore_index)
